# Initial kernel scaffold; baseline (speedup 1.0000x reference)
import jax
import jax.numpy as jnp
from jax.experimental import pallas as pl


def kernel(x, edge_index, batch, conv_W, conv_b, bn_g, bn_b, bno_g, bno_b, W1, b1, W2, b2, W3, b3):
    raise NotImplementedError("write your pallas kernel here")


# speedup vs baseline: 10.2242x; 10.2242x over previous
import functools

import jax
import jax.numpy as jnp
from jax import lax
from jax.experimental import pallas as pl
from jax.experimental.pallas import tpu as pltpu
from jax.experimental.pallas import tpu_sc as plsc

N = 10000
E = 320000
D = 128
B = 64
D_CAT = 3 * D
D_POOL = 3 * D_CAT
EPS = 1e-5

NC = 2
NS = 16
NW = NC * NS
EPW = E // NW
CH = 80
NCHUNK = EPW // CH
RQ = 624
REM = N - NS * RQ
REM_OFF = NS * RQ

_mesh = plsc.VectorSubcoreMesh(core_axis_name="c", subcore_axis_name="s")


@functools.partial(
    pl.kernel,
    out_type=jax.ShapeDtypeStruct((NC, N, 16), jnp.float32),
    mesh=_mesh,
    scratch_types=[
        pltpu.VMEM((CH,), jnp.int32),
        pltpu.VMEM((CH, 16), jnp.float32),
        pltpu.VMEM_SHARED((N, 16), jnp.float32),
        pltpu.SemaphoreType.DMA,
    ],
    compiler_params=pltpu.CompilerParams(use_tc_tiling_on_sc=False),
)
def _sc_deg(dst_hbm, zeros16_hbm, ones16_hbm, out_hbm,
            dst_v, ones_v, acc_sh, sem):
    cid = lax.axis_index("c")
    sid = lax.axis_index("s")
    wid = sid * NC + cid
    pltpu.sync_copy(ones16_hbm, ones_v)
    pltpu.sync_copy(zeros16_hbm, acc_sh.at[pl.ds(sid * RQ, RQ)])

    @pl.when(sid == NS - 1)
    def _():
        pltpu.sync_copy(zeros16_hbm.at[pl.ds(0, REM)],
                        acc_sh.at[pl.ds(REM_OFF, REM)])

    plsc.subcore_barrier()

    def body(t, carry):
        off = wid * EPW + t * CH
        pltpu.sync_copy(dst_hbm.at[pl.ds(off, CH)], dst_v)
        pltpu.sync_copy(ones_v, acc_sh.at[dst_v], add=True)
        return carry

    lax.fori_loop(0, NCHUNK, body, 0)
    plsc.subcore_barrier()
    pltpu.sync_copy(acc_sh.at[pl.ds(sid * RQ, RQ)],
                    out_hbm.at[cid, pl.ds(sid * RQ, RQ)])

    @pl.when(sid == NS - 1)
    def _():
        pltpu.sync_copy(acc_sh.at[pl.ds(REM_OFF, REM)],
                        out_hbm.at[cid, pl.ds(REM_OFF, REM)])


@functools.partial(
    pl.kernel,
    out_type=jax.ShapeDtypeStruct((NC, N, D), jnp.float32),
    mesh=_mesh,
    scratch_types=[
        pltpu.VMEM((CH,), jnp.int32),
        pltpu.VMEM((CH,), jnp.int32),
        pltpu.VMEM((CH, D), jnp.float32),
        pltpu.VMEM_SHARED((N, D), jnp.float32),
        pltpu.SemaphoreType.DMA,
    ],
)
def _sc_agg(hws_hbm, src_hbm, dst_hbm, zerosd_hbm, out_hbm,
            src_v, dst_v, rows_v, acc_sh, sem):
    cid = lax.axis_index("c")
    sid = lax.axis_index("s")
    wid = sid * NC + cid
    pltpu.sync_copy(zerosd_hbm, acc_sh.at[pl.ds(sid * RQ, RQ)])

    @pl.when(sid == NS - 1)
    def _():
        pltpu.sync_copy(zerosd_hbm.at[pl.ds(0, REM)],
                        acc_sh.at[pl.ds(REM_OFF, REM)])

    plsc.subcore_barrier()

    def body(t, carry):
        off = wid * EPW + t * CH
        pltpu.sync_copy(src_hbm.at[pl.ds(off, CH)], src_v)
        pltpu.sync_copy(dst_hbm.at[pl.ds(off, CH)], dst_v)
        pltpu.async_copy(hws_hbm.at[src_v], rows_v, sem).wait()
        pltpu.sync_copy(rows_v, acc_sh.at[dst_v], add=True)
        return carry

    lax.fori_loop(0, NCHUNK, body, 0)
    plsc.subcore_barrier()
    pltpu.sync_copy(acc_sh.at[pl.ds(sid * RQ, RQ)],
                    out_hbm.at[cid, pl.ds(sid * RQ, RQ)])

    @pl.when(sid == NS - 1)
    def _():
        pltpu.sync_copy(acc_sh.at[pl.ds(REM_OFF, REM)],
                        out_hbm.at[cid, pl.ds(REM_OFF, REM)])


def _dinv_from_degp(degp):
    deg = degp[0, :, :1] + degp[1, :, :1] + 1.0
    return lax.rsqrt(deg)


def _tc_pre_body(x_ref, w_ref, degp_ref, hws_ref):
    dinv = _dinv_from_degp(degp_ref[...])
    hw = jnp.dot(x_ref[...], w_ref[...], preferred_element_type=jnp.float32)
    hws_ref[...] = hw * dinv


def _tc_layer_body(sp_ref, hws_ref, degp_ref, b_ref, g_ref, bb_ref, wn_ref,
                   h_ref, hwsn_ref):
    dinv = _dinv_from_degp(degp_ref[...])
    s = sp_ref[0] + sp_ref[1]
    pre = dinv * (s + hws_ref[...]) + b_ref[...]
    mu = jnp.mean(pre, axis=0, keepdims=True)
    var = jnp.mean((pre - mu) ** 2, axis=0, keepdims=True)
    h = (pre - mu) * lax.rsqrt(var + EPS) * g_ref[...] + bb_ref[...]
    h_ref[...] = h
    if hwsn_ref is not None:
        hwsn_ref[...] = jnp.dot(
            h, wn_ref[...], preferred_element_type=jnp.float32) * dinv


def _tc_layer_last_body(sp_ref, hws_ref, degp_ref, b_ref, g_ref, bb_ref,
                        h_ref):
    _tc_layer_body(sp_ref, hws_ref, degp_ref, b_ref, g_ref, bb_ref, None,
                   h_ref, None)


def _tc_head_body(h1_ref, h2_ref, h3_ref, batch_ref, bg_ref, bb_ref,
                  w1_ref, b1_ref, w2_ref, b2_ref, w3_ref, b3_ref,
                  out_ref, maxs_ref):
    h = jnp.concatenate([h1_ref[...], h2_ref[...], h3_ref[...]], axis=1)
    bcol = batch_ref[...]
    seg = lax.broadcasted_iota(jnp.int32, (N, B), 1)
    m = (bcol == seg).astype(jnp.float32)
    cnt = jnp.sum(m, axis=0)
    sums = lax.dot_general(m, h, (((0,), (0,)), ((), ())),
                           preferred_element_type=jnp.float32)

    def mbody(bi, carry):
        mask = bcol == bi
        mx = jnp.max(jnp.where(mask, h, -jnp.inf), axis=0)
        maxs_ref[pl.ds(bi, 1), :] = mx[None]
        return carry

    lax.fori_loop(0, B, mbody, 0)
    maxs = maxs_ref[...]

    avg = sums / jnp.maximum(cnt, 1.0)[:, None]
    hp = jnp.concatenate([avg, sums, maxs], axis=1)
    mu = jnp.mean(hp, axis=0, keepdims=True)
    var = jnp.mean((hp - mu) ** 2, axis=0, keepdims=True)
    z = (hp - mu) * lax.rsqrt(var + EPS) * bg_ref[...] + bb_ref[...]
    z = jnp.maximum(
        jnp.dot(z, w1_ref[...], preferred_element_type=jnp.float32)
        + b1_ref[...], 0.0)
    z = jnp.maximum(
        jnp.dot(z, w2_ref[...], preferred_element_type=jnp.float32)
        + b2_ref[...], 0.0)
    zl = jnp.dot(z, w3_ref[...], preferred_element_type=jnp.float32) \
        + b3_ref[...]
    zmax = jnp.max(zl, axis=1, keepdims=True)
    lse = zmax + jnp.log(jnp.sum(jnp.exp(zl - zmax), axis=1, keepdims=True))
    out_ref[...] = zl - lse


_f32 = jnp.float32


def _call_tc(body, out_shape, *args, scratch_shapes=()):
    return pl.pallas_call(
        body,
        out_shape=out_shape,
        scratch_shapes=list(scratch_shapes),
    )(*args)


def kernel(x, edge_index, batch, conv_W, conv_b, bn_g, bn_b, bno_g, bno_b,
           W1, b1, W2, b2, W3, b3):
    src = edge_index[0]
    dst = edge_index[1]
    zeros16 = jnp.zeros((RQ, 16), _f32)
    ones16 = jnp.ones((CH, 16), _f32)
    zerosd = jnp.zeros((RQ, D), _f32)

    degp = _sc_deg(dst, zeros16, ones16)

    hws = _call_tc(_tc_pre_body, jax.ShapeDtypeStruct((N, D), _f32),
                   x, conv_W[0], degp)

    hs = []
    for i in range(3):
        sp = _sc_agg(hws, src, dst, zerosd)
        bi = conv_b[i].reshape(1, D)
        gi = bn_g[i].reshape(1, D)
        bbi = bn_b[i].reshape(1, D)
        if i < 2:
            h, hws = _call_tc(
                _tc_layer_body,
                (jax.ShapeDtypeStruct((N, D), _f32),
                 jax.ShapeDtypeStruct((N, D), _f32)),
                sp, hws, degp, bi, gi, bbi, conv_W[i + 1])
        else:
            h = _call_tc(
                _tc_layer_last_body, jax.ShapeDtypeStruct((N, D), _f32),
                sp, hws, degp, bi, gi, bbi)
        hs.append(h)

    out = _call_tc(
        _tc_head_body, jax.ShapeDtypeStruct((B, 10), _f32),
        hs[0], hs[1], hs[2], batch.reshape(N, 1),
        bno_g.reshape(1, D_POOL), bno_b.reshape(1, D_POOL),
        W1, b1.reshape(1, -1), W2, b2.reshape(1, -1), W3, b3.reshape(1, -1),
        scratch_shapes=[pltpu.VMEM((B, D_CAT), _f32)])
    return out


# speedup vs baseline: 16.3428x; 1.5984x over previous
import functools

import jax
import jax.numpy as jnp
from jax import lax
from jax.experimental import pallas as pl
from jax.experimental.pallas import tpu as pltpu
from jax.experimental.pallas import tpu_sc as plsc

N = 10000
E = 320000
D = 128
B = 64
D_CAT = 3 * D
D_POOL = 3 * D_CAT
EPS = 1e-5

NC = 2
NS = 16
NW = NC * NS
EPW = E // NW
CH = 80
NCHUNK = EPW // CH
RQ = 624
REM = N - NS * RQ
REM_OFF = NS * RQ

_mesh = plsc.VectorSubcoreMesh(core_axis_name="c", subcore_axis_name="s")


@functools.partial(
    pl.kernel,
    out_type=jax.ShapeDtypeStruct((NC, N, 16), jnp.float32),
    mesh=_mesh,
    scratch_types=[
        pltpu.VMEM((CH,), jnp.int32),
        pltpu.VMEM((CH, 16), jnp.float32),
        pltpu.VMEM_SHARED((N, 16), jnp.float32),
        pltpu.SemaphoreType.DMA,
    ],
    compiler_params=pltpu.CompilerParams(use_tc_tiling_on_sc=False),
)
def _sc_deg(dst_hbm, zeros16_hbm, ones16_hbm, out_hbm,
            dst_v, ones_v, acc_sh, sem):
    cid = lax.axis_index("c")
    sid = lax.axis_index("s")
    wid = sid * NC + cid
    pltpu.sync_copy(ones16_hbm, ones_v)
    pltpu.sync_copy(zeros16_hbm, acc_sh.at[pl.ds(sid * RQ, RQ)])

    @pl.when(sid == NS - 1)
    def _():
        pltpu.sync_copy(zeros16_hbm.at[pl.ds(0, REM)],
                        acc_sh.at[pl.ds(REM_OFF, REM)])

    plsc.subcore_barrier()

    def body(t, carry):
        off = wid * EPW + t * CH
        pltpu.sync_copy(dst_hbm.at[pl.ds(off, CH)], dst_v)
        pltpu.sync_copy(ones_v, acc_sh.at[dst_v], add=True)
        return carry

    lax.fori_loop(0, NCHUNK, body, 0)
    plsc.subcore_barrier()
    pltpu.sync_copy(acc_sh.at[pl.ds(sid * RQ, RQ)],
                    out_hbm.at[cid, pl.ds(sid * RQ, RQ)])

    @pl.when(sid == NS - 1)
    def _():
        pltpu.sync_copy(acc_sh.at[pl.ds(REM_OFF, REM)],
                        out_hbm.at[cid, pl.ds(REM_OFF, REM)])


@functools.partial(
    pl.kernel,
    out_type=jax.ShapeDtypeStruct((NC, N, D), jnp.float32),
    mesh=_mesh,
    scratch_types=[
        pltpu.VMEM((EPW,), jnp.int32),
        pltpu.VMEM((EPW,), jnp.int32),
        pltpu.VMEM((CH, D), jnp.float32),
        pltpu.VMEM((CH, D), jnp.float32),
        pltpu.VMEM_SHARED((N, D), jnp.float32),
        pltpu.SemaphoreType.DMA,
        pltpu.SemaphoreType.DMA,
        pltpu.SemaphoreType.DMA,
        pltpu.SemaphoreType.DMA,
    ],
    compiler_params=pltpu.CompilerParams(use_tc_tiling_on_sc=False),
)
def _sc_agg(hws_hbm, src_hbm, dst_hbm, zerosd_hbm, out_hbm,
            src_all, dst_all, rows_a, rows_b, acc_sh, sg_a, sg_b, ss_a, ss_b):
    cid = lax.axis_index("c")
    sid = lax.axis_index("s")
    wid = sid * NC + cid
    base = wid * EPW
    pltpu.sync_copy(zerosd_hbm, acc_sh.at[pl.ds(sid * RQ, RQ)])

    @pl.when(sid == NS - 1)
    def _():
        pltpu.sync_copy(zerosd_hbm.at[pl.ds(0, REM)],
                        acc_sh.at[pl.ds(REM_OFF, REM)])

    pltpu.sync_copy(src_hbm.at[pl.ds(base, EPW)], src_all)
    pltpu.sync_copy(dst_hbm.at[pl.ds(base, EPW)], dst_all)
    plsc.subcore_barrier()

    def gissue(t, rows, sem):
        pltpu.async_copy(hws_hbm.at[src_all.at[pl.ds(t * CH, CH)]], rows, sem)

    def gwait(rows, sem):
        pltpu.make_async_copy(
            hws_hbm.at[src_all.at[pl.ds(0, CH)]], rows, sem).wait()

    def sissue(t, rows, sem):
        pltpu.async_copy(rows, acc_sh.at[dst_all.at[pl.ds(t * CH, CH)]], sem,
                         add=True)

    def swait(rows, sem):
        pltpu.make_async_copy(
            rows, acc_sh.at[dst_all.at[pl.ds(0, CH)]], sem).wait()

    gissue(0, rows_a, sg_a)

    def body(t, carry):
        @pl.when(t % 2 == 0)
        def _():
            gwait(rows_a, sg_a)
            sissue(t, rows_a, ss_a)

            @pl.when(t + 1 < NCHUNK)
            def _():
                @pl.when(t > 0)
                def _():
                    swait(rows_b, ss_b)
                gissue(t + 1, rows_b, sg_b)

        @pl.when(t % 2 == 1)
        def _():
            gwait(rows_b, sg_b)
            sissue(t, rows_b, ss_b)

            @pl.when(t + 1 < NCHUNK)
            def _():
                swait(rows_a, ss_a)
                gissue(t + 1, rows_a, sg_a)

        return carry

    lax.fori_loop(0, NCHUNK, body, 0)
    swait(rows_b, ss_b)
    swait(rows_a, ss_a)
    plsc.subcore_barrier()
    pltpu.sync_copy(acc_sh.at[pl.ds(sid * RQ, RQ)],
                    out_hbm.at[cid, pl.ds(sid * RQ, RQ)])

    @pl.when(sid == NS - 1)
    def _():
        pltpu.sync_copy(acc_sh.at[pl.ds(REM_OFF, REM)],
                        out_hbm.at[cid, pl.ds(REM_OFF, REM)])


def _dinv_from_degp(degp):
    deg = degp[0, :, :1] + degp[1, :, :1] + 1.0
    return lax.rsqrt(deg)


def _tc_pre_body(x_ref, w_ref, degp_ref, hws_ref):
    dinv = _dinv_from_degp(degp_ref[...])
    hw = jnp.dot(x_ref[...], w_ref[...], preferred_element_type=jnp.float32)
    hws_ref[...] = hw * dinv


def _tc_layer_body(sp_ref, hws_ref, degp_ref, b_ref, g_ref, bb_ref, wn_ref,
                   h_ref, hwsn_ref):
    dinv = _dinv_from_degp(degp_ref[...])
    s = sp_ref[0] + sp_ref[1]
    pre = dinv * (s + hws_ref[...]) + b_ref[...]
    mu = jnp.mean(pre, axis=0, keepdims=True)
    var = jnp.mean((pre - mu) ** 2, axis=0, keepdims=True)
    h = (pre - mu) * lax.rsqrt(var + EPS) * g_ref[...] + bb_ref[...]
    h_ref[...] = h
    if hwsn_ref is not None:
        hwsn_ref[...] = jnp.dot(
            h, wn_ref[...], preferred_element_type=jnp.float32) * dinv


def _tc_layer_last_body(sp_ref, hws_ref, degp_ref, b_ref, g_ref, bb_ref,
                        h_ref):
    _tc_layer_body(sp_ref, hws_ref, degp_ref, b_ref, g_ref, bb_ref, None,
                   h_ref, None)


def _tc_head_body(h1_ref, h2_ref, h3_ref, batch_ref, bg_ref, bb_ref,
                  w1_ref, b1_ref, w2_ref, b2_ref, w3_ref, b3_ref,
                  out_ref, maxs_ref):
    h = jnp.concatenate([h1_ref[...], h2_ref[...], h3_ref[...]], axis=1)
    bcol = batch_ref[...]
    seg = lax.broadcasted_iota(jnp.int32, (N, B), 1)
    m = (bcol == seg).astype(jnp.float32)
    cnt = jnp.sum(m, axis=0)
    sums = lax.dot_general(m, h, (((0,), (0,)), ((), ())),
                           preferred_element_type=jnp.float32)

    def mbody(bi, carry):
        mask = bcol == bi
        mx = jnp.max(jnp.where(mask, h, -jnp.inf), axis=0)
        maxs_ref[pl.ds(bi, 1), :] = mx[None]
        return carry

    lax.fori_loop(0, B, mbody, 0)
    maxs = maxs_ref[...]

    avg = sums / jnp.maximum(cnt, 1.0)[:, None]
    hp = jnp.concatenate([avg, sums, maxs], axis=1)
    mu = jnp.mean(hp, axis=0, keepdims=True)
    var = jnp.mean((hp - mu) ** 2, axis=0, keepdims=True)
    z = (hp - mu) * lax.rsqrt(var + EPS) * bg_ref[...] + bb_ref[...]
    z = jnp.maximum(
        jnp.dot(z, w1_ref[...], preferred_element_type=jnp.float32)
        + b1_ref[...], 0.0)
    z = jnp.maximum(
        jnp.dot(z, w2_ref[...], preferred_element_type=jnp.float32)
        + b2_ref[...], 0.0)
    zl = jnp.dot(z, w3_ref[...], preferred_element_type=jnp.float32) \
        + b3_ref[...]
    zmax = jnp.max(zl, axis=1, keepdims=True)
    lse = zmax + jnp.log(jnp.sum(jnp.exp(zl - zmax), axis=1, keepdims=True))
    out_ref[...] = zl - lse


_f32 = jnp.float32


def _call_tc(body, out_shape, *args, scratch_shapes=()):
    return pl.pallas_call(
        body,
        out_shape=out_shape,
        scratch_shapes=list(scratch_shapes),
    )(*args)


def kernel(x, edge_index, batch, conv_W, conv_b, bn_g, bn_b, bno_g, bno_b,
           W1, b1, W2, b2, W3, b3):
    src = edge_index[0]
    dst = edge_index[1]
    zeros16 = jnp.zeros((RQ, 16), _f32)
    ones16 = jnp.ones((CH, 16), _f32)
    zerosd = jnp.zeros((RQ, D), _f32)

    degp = _sc_deg(dst, zeros16, ones16)

    hws = _call_tc(_tc_pre_body, jax.ShapeDtypeStruct((N, D), _f32),
                   x, conv_W[0], degp)

    hs = []
    for i in range(3):
        sp = _sc_agg(hws, src, dst, zerosd)
        bi = conv_b[i].reshape(1, D)
        gi = bn_g[i].reshape(1, D)
        bbi = bn_b[i].reshape(1, D)
        if i < 2:
            h, hws = _call_tc(
                _tc_layer_body,
                (jax.ShapeDtypeStruct((N, D), _f32),
                 jax.ShapeDtypeStruct((N, D), _f32)),
                sp, hws, degp, bi, gi, bbi, conv_W[i + 1])
        else:
            h = _call_tc(
                _tc_layer_last_body, jax.ShapeDtypeStruct((N, D), _f32),
                sp, hws, degp, bi, gi, bbi)
        hs.append(h)

    out = _call_tc(
        _tc_head_body, jax.ShapeDtypeStruct((B, 10), _f32),
        hs[0], hs[1], hs[2], batch.reshape(N, 1),
        bno_g.reshape(1, D_POOL), bno_b.reshape(1, D_POOL),
        W1, b1.reshape(1, -1), W2, b2.reshape(1, -1), W3, b3.reshape(1, -1),
        scratch_shapes=[pltpu.VMEM((B, D_CAT), _f32)])
    return out


# speedup vs baseline: 27.8929x; 1.7067x over previous
import functools

import jax
import jax.numpy as jnp
from jax import lax
from jax.experimental import pallas as pl
from jax.experimental.pallas import tpu as pltpu
from jax.experimental.pallas import tpu_sc as plsc

N = 10000
E = 320000
D = 128
B = 64
D_CAT = 3 * D
D_POOL = 3 * D_CAT
EPS = 1e-5

NC = 2
NS = 16
NW = NC * NS
EPW = E // NW
CH = 80
NCHUNK = EPW // CH
RQ = 624
REM = N - NS * RQ
REM_OFF = NS * RQ

_mesh = plsc.VectorSubcoreMesh(core_axis_name="c", subcore_axis_name="s")


@functools.partial(
    pl.kernel,
    out_type=jax.ShapeDtypeStruct((NC, N, 16), jnp.float32),
    mesh=_mesh,
    scratch_types=[
        pltpu.VMEM((EPW,), jnp.int32),
        pltpu.VMEM((CH, 16), jnp.float32),
        pltpu.VMEM_SHARED((N, 16), jnp.float32),
        pltpu.SemaphoreType.DMA,
    ],
    compiler_params=pltpu.CompilerParams(use_tc_tiling_on_sc=False),
)
def _sc_deg(edge_hbm, zeros16_hbm, ones16_hbm, out_hbm,
            dst_all, ones_v, acc_sh, sem):
    cid = lax.axis_index("c")
    sid = lax.axis_index("s")
    wid = sid * NC + cid
    pltpu.sync_copy(ones16_hbm, ones_v)
    pltpu.sync_copy(edge_hbm.at[1, pl.ds(wid * EPW, EPW)], dst_all)
    pltpu.sync_copy(zeros16_hbm, acc_sh.at[pl.ds(sid * RQ, RQ)])

    @pl.when(sid == NS - 1)
    def _():
        pltpu.sync_copy(zeros16_hbm.at[pl.ds(0, REM)],
                        acc_sh.at[pl.ds(REM_OFF, REM)])

    plsc.subcore_barrier()

    def body(t, carry):
        pltpu.async_copy(ones_v, acc_sh.at[dst_all.at[pl.ds(t * CH, CH)]],
                         sem, add=True)
        return carry

    lax.fori_loop(0, NCHUNK, body, 0)

    def drain(t, carry):
        pltpu.make_async_copy(
            ones_v, acc_sh.at[dst_all.at[pl.ds(0, CH)]], sem).wait()
        return carry

    lax.fori_loop(0, NCHUNK, drain, 0)
    plsc.subcore_barrier()
    pltpu.sync_copy(acc_sh.at[pl.ds(sid * RQ, RQ)],
                    out_hbm.at[cid, pl.ds(sid * RQ, RQ)])

    @pl.when(sid == NS - 1)
    def _():
        pltpu.sync_copy(acc_sh.at[pl.ds(REM_OFF, REM)],
                        out_hbm.at[cid, pl.ds(REM_OFF, REM)])


@functools.partial(
    pl.kernel,
    out_type=jax.ShapeDtypeStruct((NC, N, D), jnp.float32),
    mesh=_mesh,
    scratch_types=[
        pltpu.VMEM((EPW,), jnp.int32),
        pltpu.VMEM((EPW,), jnp.int32),
        pltpu.VMEM((CH, D), jnp.float32),
        pltpu.VMEM((CH, D), jnp.float32),
        pltpu.VMEM((CH, D), jnp.float32),
        pltpu.VMEM_SHARED((N, D), jnp.float32),
        pltpu.SemaphoreType.DMA,
        pltpu.SemaphoreType.DMA,
        pltpu.SemaphoreType.DMA,
        pltpu.SemaphoreType.DMA,
        pltpu.SemaphoreType.DMA,
        pltpu.SemaphoreType.DMA,
    ],
    compiler_params=pltpu.CompilerParams(use_tc_tiling_on_sc=False),
)
def _sc_agg(hws_hbm, edge_hbm, zerosd_hbm, out_hbm,
            src_all, dst_all, rows_a, rows_b, rows_c, acc_sh,
            sg_a, sg_b, sg_c, ss_a, ss_b, ss_c):
    cid = lax.axis_index("c")
    sid = lax.axis_index("s")
    wid = sid * NC + cid
    base = wid * EPW
    pltpu.sync_copy(zerosd_hbm, acc_sh.at[pl.ds(sid * RQ, RQ)])

    @pl.when(sid == NS - 1)
    def _():
        pltpu.sync_copy(zerosd_hbm.at[pl.ds(0, REM)],
                        acc_sh.at[pl.ds(REM_OFF, REM)])

    pltpu.sync_copy(edge_hbm.at[0, pl.ds(base, EPW)], src_all)
    pltpu.sync_copy(edge_hbm.at[1, pl.ds(base, EPW)], dst_all)
    plsc.subcore_barrier()

    def gissue(t, rows, sem):
        pltpu.async_copy(hws_hbm.at[src_all.at[pl.ds(t * CH, CH)]], rows, sem)

    def gwait(rows, sem):
        pltpu.make_async_copy(
            hws_hbm.at[src_all.at[pl.ds(0, CH)]], rows, sem).wait()

    def sissue(t, rows, sem):
        pltpu.async_copy(rows, acc_sh.at[dst_all.at[pl.ds(t * CH, CH)]], sem,
                         add=True)

    def swait(rows, sem):
        pltpu.make_async_copy(
            rows, acc_sh.at[dst_all.at[pl.ds(0, CH)]], sem).wait()

    gissue(0, rows_a, sg_a)
    gissue(1, rows_b, sg_b)

    bufs = ((rows_a, sg_a, ss_a), (rows_b, sg_b, ss_b), (rows_c, sg_c, ss_c))

    def body(t, carry):
        for r in range(3):
            rows_p, sg_p, ss_p = bufs[r]
            rows_q, sg_q, ss_q = bufs[(r + 2) % 3]

            @pl.when(t % 3 == r)
            def _(rows_p=rows_p, sg_p=sg_p, ss_p=ss_p,
                  rows_q=rows_q, sg_q=sg_q, ss_q=ss_q):
                gwait(rows_p, sg_p)
                sissue(t, rows_p, ss_p)

                @pl.when(t + 2 < NCHUNK)
                def _():
                    @pl.when(t > 0)
                    def _():
                        swait(rows_q, ss_q)
                    gissue(t + 2, rows_q, sg_q)

        return carry

    lax.fori_loop(0, NCHUNK, body, 0)
    swait(rows_b, ss_b)
    swait(rows_a, ss_a)
    swait(rows_c, ss_c)
    plsc.subcore_barrier()
    pltpu.sync_copy(acc_sh.at[pl.ds(sid * RQ, RQ)],
                    out_hbm.at[cid, pl.ds(sid * RQ, RQ)])

    @pl.when(sid == NS - 1)
    def _():
        pltpu.sync_copy(acc_sh.at[pl.ds(REM_OFF, REM)],
                        out_hbm.at[cid, pl.ds(REM_OFF, REM)])


BPW = B // NW
CHP = 64
NOFF = 80


@functools.partial(
    pl.kernel,
    out_type=(jax.ShapeDtypeStruct((B, D_CAT), jnp.float32),
              jax.ShapeDtypeStruct((B, D_CAT), jnp.float32)),
    mesh=_mesh,
    scratch_types=[
        pltpu.VMEM((NOFF,), jnp.int32),
        pltpu.VMEM((CHP, D), jnp.float32),
        pltpu.VMEM((D_CAT,), jnp.float32),
        pltpu.VMEM((D_CAT,), jnp.float32),
        pltpu.SemaphoreType.DMA,
    ],
    compiler_params=pltpu.CompilerParams(use_tc_tiling_on_sc=False,
                                         needs_layout_passes=False),
)
def _sc_pool(h1_hbm, h2_hbm, h3_hbm, offs_hbm, sums_hbm, maxs_hbm,
             offs_v, hbuf_v, sstage_v, mstage_v, sem):
    cid = lax.axis_index("c")
    sid = lax.axis_index("s")
    wid = sid * NC + cid
    pltpu.sync_copy(offs_hbm, offs_v)

    def sel(k):
        acc = jnp.int32(0)
        for j in range(NOFF // 16):
            v = offs_v[pl.ds(j * 16, 16)]
            idx = lax.iota(jnp.int32, 16) + j * 16
            acc = acc + jnp.sum(jnp.where(idx == k, v, jnp.int32(0)))
        return acc

    for gi in range(BPW):
        g = wid * BPW + gi
        start = sel(g)
        end = sel(g + 1)
        nch = (end - start + CHP - 1) // CHP
        for ti, h_hbm in enumerate((h1_hbm, h2_hbm, h3_hbm)):
            def chunk_body(c, carry, h_hbm=h_hbm):
                row0 = start + c * CHP
                base = jnp.minimum(row0, N - CHP)
                jlo = row0 - base
                jhi = jnp.minimum(end, base + CHP) - base
                pltpu.sync_copy(h_hbm.at[pl.ds(base, CHP)], hbuf_v)

                def row_body(j, rc):
                    sums = []
                    maxs = []
                    for k in range(8):
                        v = hbuf_v[j, pl.ds(k * 16, 16)]
                        sums.append(rc[k] + v)
                        maxs.append(jnp.maximum(rc[8 + k], v))
                    return tuple(sums) + tuple(maxs)

                return lax.fori_loop(jlo, jhi, row_body, carry)

            init = tuple(jnp.zeros((16,), jnp.float32) for _ in range(8)) \
                + tuple(jnp.full((16,), -jnp.inf, jnp.float32)
                        for _ in range(8))
            accs = lax.fori_loop(0, nch, chunk_body, init)
            for k in range(8):
                sstage_v[pl.ds(ti * D + k * 16, 16)] = accs[k]
                mstage_v[pl.ds(ti * D + k * 16, 16)] = accs[8 + k]
        pltpu.sync_copy(sstage_v, sums_hbm.at[g])
        pltpu.sync_copy(mstage_v, maxs_hbm.at[g])


def _dinv_from_degp(degp):
    deg = degp[0, :, :1] + degp[1, :, :1] + 1.0
    return lax.rsqrt(deg)


def _tc_pre_body(x_ref, w_ref, degp_ref, hws_ref):
    dinv = _dinv_from_degp(degp_ref[...])
    hw = jnp.dot(x_ref[...], w_ref[...], preferred_element_type=jnp.float32)
    hws_ref[...] = hw * dinv


def _tc_layer_body(sp_ref, hws_ref, degp_ref, b_ref, g_ref, bb_ref, wn_ref,
                   h_ref, hwsn_ref):
    dinv = _dinv_from_degp(degp_ref[...])
    s = sp_ref[0] + sp_ref[1]
    pre = dinv * (s + hws_ref[...]) + b_ref[...]
    mu = jnp.mean(pre, axis=0, keepdims=True)
    var = jnp.mean((pre - mu) ** 2, axis=0, keepdims=True)
    h = (pre - mu) * lax.rsqrt(var + EPS) * g_ref[...] + bb_ref[...]
    h_ref[...] = h
    if hwsn_ref is not None:
        hwsn_ref[...] = jnp.dot(
            h, wn_ref[...], preferred_element_type=jnp.float32) * dinv


def _tc_layer_last_body(sp_ref, hws_ref, degp_ref, b_ref, g_ref, bb_ref,
                        batch_ref, h_ref, offs_ref):
    _tc_layer_body(sp_ref, hws_ref, degp_ref, b_ref, g_ref, bb_ref, None,
                   h_ref, None)
    bcol = batch_ref[...]
    seg = lax.broadcasted_iota(jnp.int32, (N, B), 1)
    m = (bcol == seg).astype(jnp.float32)
    cnt = jnp.sum(m, axis=0)
    tri = (lax.broadcasted_iota(jnp.int32, (B, B), 0)
           < lax.broadcasted_iota(jnp.int32, (B, B), 1)).astype(jnp.float32)
    offs = lax.dot_general(cnt, tri, (((0,), (0,)), ((), ())),
                           preferred_element_type=jnp.float32)
    offs_full = jnp.concatenate(
        [offs, jnp.full((NOFF - B,), float(N), jnp.float32)])
    offs_ref[...] = offs_full.astype(jnp.int32)


def _tc_head_body(sums_ref, maxs_ref, offs_ref, bg_ref, bbo_ref,
                  w1_ref, b1_ref, w2_ref, b2_ref, w3_ref, b3_ref,
                  out_ref):
    sums = sums_ref[...]
    maxs = maxs_ref[...]
    offs = offs_ref[...].astype(jnp.float32)
    cnt = offs[1:B + 1] - offs[0:B]
    avg = sums / jnp.maximum(cnt, 1.0)[:, None]
    hp = jnp.concatenate([avg, sums, maxs], axis=1)
    mu = jnp.mean(hp, axis=0, keepdims=True)
    var = jnp.mean((hp - mu) ** 2, axis=0, keepdims=True)
    z = (hp - mu) * lax.rsqrt(var + EPS) * bg_ref[...] + bbo_ref[...]
    z = jnp.maximum(
        jnp.dot(z, w1_ref[...], preferred_element_type=jnp.float32)
        + b1_ref[...], 0.0)
    z = jnp.maximum(
        jnp.dot(z, w2_ref[...], preferred_element_type=jnp.float32)
        + b2_ref[...], 0.0)
    zl = jnp.dot(z, w3_ref[...], preferred_element_type=jnp.float32) \
        + b3_ref[...]
    zmax = jnp.max(zl, axis=1, keepdims=True)
    lse = zmax + jnp.log(jnp.sum(jnp.exp(zl - zmax), axis=1, keepdims=True))
    out_ref[...] = zl - lse


_f32 = jnp.float32


def _call_tc(body, out_shape, *args, scratch_shapes=()):
    return pl.pallas_call(
        body,
        out_shape=out_shape,
        scratch_shapes=list(scratch_shapes),
    )(*args)


def kernel(x, edge_index, batch, conv_W, conv_b, bn_g, bn_b, bno_g, bno_b,
           W1, b1, W2, b2, W3, b3):
    zeros16 = jnp.zeros((RQ, 16), _f32)
    ones16 = jnp.ones((CH, 16), _f32)
    zerosd = jnp.zeros((RQ, D), _f32)

    degp = _sc_deg(edge_index, zeros16, ones16)

    hws = _call_tc(_tc_pre_body, jax.ShapeDtypeStruct((N, D), _f32),
                   x, conv_W[0], degp)

    hs = []
    for i in range(2):
        sp = _sc_agg(hws, edge_index, zerosd)
        h, hws = _call_tc(
            _tc_layer_body,
            (jax.ShapeDtypeStruct((N, D), _f32),
             jax.ShapeDtypeStruct((N, D), _f32)),
            sp, hws, degp, conv_b[i].reshape(1, D), bn_g[i].reshape(1, D),
            bn_b[i].reshape(1, D), conv_W[i + 1])
        hs.append(h)

    sp = _sc_agg(hws, edge_index, zerosd)
    h3, offs = _call_tc(
        _tc_layer_last_body,
        (jax.ShapeDtypeStruct((N, D), _f32),
         jax.ShapeDtypeStruct((NOFF,), jnp.int32)),
        sp, hws, degp, conv_b[2].reshape(1, D), bn_g[2].reshape(1, D),
        bn_b[2].reshape(1, D), batch.reshape(N, 1))
    sums, maxs = _sc_pool(hs[0], hs[1], h3, offs)
    out = _call_tc(
        _tc_head_body, jax.ShapeDtypeStruct((B, 10), _f32),
        sums, maxs, offs,
        bno_g.reshape(1, D_POOL), bno_b.reshape(1, D_POOL),
        W1, b1.reshape(1, -1), W2, b2.reshape(1, -1), W3, b3.reshape(1, -1))
    return out


# speedup vs baseline: 28.5664x; 1.0241x over previous
import functools

import jax
import jax.numpy as jnp
from jax import lax
from jax.experimental import pallas as pl
from jax.experimental.pallas import tpu as pltpu
from jax.experimental.pallas import tpu_sc as plsc

N = 10000
E = 320000
D = 128
B = 64
D_CAT = 3 * D
D_POOL = 3 * D_CAT
EPS = 1e-5

NC = 2
NS = 16
NW = NC * NS
EPW = E // NW
CH = 80
NCHUNK = EPW // CH
RQ = 624
REM = N - NS * RQ
REM_OFF = NS * RQ

_mesh = plsc.VectorSubcoreMesh(core_axis_name="c", subcore_axis_name="s")


@functools.partial(
    pl.kernel,
    out_type=jax.ShapeDtypeStruct((NC, N, 16), jnp.float32),
    mesh=_mesh,
    scratch_types=[
        pltpu.VMEM((EPW,), jnp.int32),
        pltpu.VMEM((CH, 16), jnp.float32),
        pltpu.VMEM_SHARED((N, 16), jnp.float32),
        pltpu.SemaphoreType.DMA,
    ],
    compiler_params=pltpu.CompilerParams(use_tc_tiling_on_sc=False),
)
def _sc_deg(edge_hbm, zeros16_hbm, ones16_hbm, out_hbm,
            dst_all, ones_v, acc_sh, sem):
    cid = lax.axis_index("c")
    sid = lax.axis_index("s")
    wid = sid * NC + cid
    pltpu.async_copy(zeros16_hbm, acc_sh.at[pl.ds(sid * RQ, RQ)], sem)

    @pl.when(sid == NS - 1)
    def _():
        pltpu.async_copy(zeros16_hbm.at[pl.ds(0, REM)],
                         acc_sh.at[pl.ds(REM_OFF, REM)], sem)

    pltpu.sync_copy(ones16_hbm, ones_v)
    pltpu.sync_copy(edge_hbm.at[1, pl.ds(wid * EPW, EPW)], dst_all)
    pltpu.make_async_copy(zeros16_hbm, acc_sh.at[pl.ds(sid * RQ, RQ)],
                          sem).wait()

    @pl.when(sid == NS - 1)
    def _():
        pltpu.make_async_copy(zeros16_hbm.at[pl.ds(0, REM)],
                              acc_sh.at[pl.ds(REM_OFF, REM)], sem).wait()

    plsc.subcore_barrier()

    def body(t, carry):
        pltpu.async_copy(ones_v, acc_sh.at[dst_all.at[pl.ds(t * CH, CH)]],
                         sem, add=True)
        return carry

    lax.fori_loop(0, NCHUNK, body, 0)

    def drain(t, carry):
        pltpu.make_async_copy(
            ones_v, acc_sh.at[dst_all.at[pl.ds(0, CH)]], sem).wait()
        return carry

    lax.fori_loop(0, NCHUNK, drain, 0)
    plsc.subcore_barrier()
    pltpu.sync_copy(acc_sh.at[pl.ds(sid * RQ, RQ)],
                    out_hbm.at[cid, pl.ds(sid * RQ, RQ)])

    @pl.when(sid == NS - 1)
    def _():
        pltpu.sync_copy(acc_sh.at[pl.ds(REM_OFF, REM)],
                        out_hbm.at[cid, pl.ds(REM_OFF, REM)])


@functools.partial(
    pl.kernel,
    out_type=jax.ShapeDtypeStruct((NC, N, D), jnp.float32),
    mesh=_mesh,
    scratch_types=[
        pltpu.VMEM((EPW,), jnp.int32),
        pltpu.VMEM((EPW,), jnp.int32),
        pltpu.VMEM((CH, D), jnp.float32),
        pltpu.VMEM((CH, D), jnp.float32),
        pltpu.VMEM((CH, D), jnp.float32),
        pltpu.VMEM_SHARED((N, D), jnp.float32),
        pltpu.SemaphoreType.DMA,
        pltpu.SemaphoreType.DMA,
        pltpu.SemaphoreType.DMA,
        pltpu.SemaphoreType.DMA,
        pltpu.SemaphoreType.DMA,
        pltpu.SemaphoreType.DMA,
        pltpu.SemaphoreType.DMA,
    ],
    compiler_params=pltpu.CompilerParams(use_tc_tiling_on_sc=False),
)
def _sc_agg(hws_hbm, edge_hbm, zerosd_hbm, out_hbm,
            src_all, dst_all, rows_a, rows_b, rows_c, acc_sh,
            sg_a, sg_b, sg_c, ss_a, ss_b, ss_c, sz):
    cid = lax.axis_index("c")
    sid = lax.axis_index("s")
    wid = sid * NC + cid
    base = wid * EPW
    pltpu.async_copy(zerosd_hbm, acc_sh.at[pl.ds(sid * RQ, RQ)], sz)

    @pl.when(sid == NS - 1)
    def _():
        pltpu.async_copy(zerosd_hbm.at[pl.ds(0, REM)],
                         acc_sh.at[pl.ds(REM_OFF, REM)], sz)

    pltpu.sync_copy(edge_hbm.at[0, pl.ds(base, EPW)], src_all)
    pltpu.sync_copy(edge_hbm.at[1, pl.ds(base, EPW)], dst_all)

    def gissue(t, rows, sem):
        pltpu.async_copy(hws_hbm.at[src_all.at[pl.ds(t * CH, CH)]], rows, sem)

    def gwait(rows, sem):
        pltpu.make_async_copy(
            hws_hbm.at[src_all.at[pl.ds(0, CH)]], rows, sem).wait()

    def sissue(t, rows, sem):
        pltpu.async_copy(rows, acc_sh.at[dst_all.at[pl.ds(t * CH, CH)]], sem,
                         add=True)

    def swait(rows, sem):
        pltpu.make_async_copy(
            rows, acc_sh.at[dst_all.at[pl.ds(0, CH)]], sem).wait()

    gissue(0, rows_a, sg_a)
    gissue(1, rows_b, sg_b)
    pltpu.make_async_copy(zerosd_hbm, acc_sh.at[pl.ds(sid * RQ, RQ)],
                          sz).wait()

    @pl.when(sid == NS - 1)
    def _():
        pltpu.make_async_copy(zerosd_hbm.at[pl.ds(0, REM)],
                              acc_sh.at[pl.ds(REM_OFF, REM)], sz).wait()

    plsc.subcore_barrier()

    bufs = ((rows_a, sg_a, ss_a), (rows_b, sg_b, ss_b), (rows_c, sg_c, ss_c))

    def body(t, carry):
        for r in range(3):
            rows_p, sg_p, ss_p = bufs[r]
            rows_q, sg_q, ss_q = bufs[(r + 2) % 3]

            @pl.when(t % 3 == r)
            def _(rows_p=rows_p, sg_p=sg_p, ss_p=ss_p,
                  rows_q=rows_q, sg_q=sg_q, ss_q=ss_q):
                gwait(rows_p, sg_p)
                sissue(t, rows_p, ss_p)

                @pl.when(t + 2 < NCHUNK)
                def _():
                    @pl.when(t > 0)
                    def _():
                        swait(rows_q, ss_q)
                    gissue(t + 2, rows_q, sg_q)

        return carry

    lax.fori_loop(0, NCHUNK, body, 0)
    swait(rows_b, ss_b)
    swait(rows_a, ss_a)
    swait(rows_c, ss_c)
    plsc.subcore_barrier()
    pltpu.sync_copy(acc_sh.at[pl.ds(sid * RQ, RQ)],
                    out_hbm.at[cid, pl.ds(sid * RQ, RQ)])

    @pl.when(sid == NS - 1)
    def _():
        pltpu.sync_copy(acc_sh.at[pl.ds(REM_OFF, REM)],
                        out_hbm.at[cid, pl.ds(REM_OFF, REM)])


BPW = B // NW
CHP = 64
NOFF = 80


@functools.partial(
    pl.kernel,
    out_type=(jax.ShapeDtypeStruct((B, D_CAT), jnp.float32),
              jax.ShapeDtypeStruct((B, D_CAT), jnp.float32)),
    mesh=_mesh,
    scratch_types=[
        pltpu.VMEM((NOFF,), jnp.int32),
        pltpu.VMEM((CHP, D), jnp.float32),
        pltpu.VMEM((D_CAT,), jnp.float32),
        pltpu.VMEM((D_CAT,), jnp.float32),
        pltpu.SemaphoreType.DMA,
    ],
    compiler_params=pltpu.CompilerParams(use_tc_tiling_on_sc=False,
                                         needs_layout_passes=False),
)
def _sc_pool(h1_hbm, h2_hbm, h3_hbm, offs_hbm, sums_hbm, maxs_hbm,
             offs_v, hbuf_v, sstage_v, mstage_v, sem):
    cid = lax.axis_index("c")
    sid = lax.axis_index("s")
    wid = sid * NC + cid
    pltpu.sync_copy(offs_hbm, offs_v)

    def sel(k):
        acc = jnp.int32(0)
        for j in range(NOFF // 16):
            v = offs_v[pl.ds(j * 16, 16)]
            idx = lax.iota(jnp.int32, 16) + j * 16
            acc = acc + jnp.sum(jnp.where(idx == k, v, jnp.int32(0)))
        return acc

    for gi in range(BPW):
        g = wid * BPW + gi
        start = sel(g)
        end = sel(g + 1)
        nch = (end - start + CHP - 1) // CHP
        for ti, h_hbm in enumerate((h1_hbm, h2_hbm, h3_hbm)):
            def chunk_body(c, carry, h_hbm=h_hbm):
                row0 = start + c * CHP
                base = jnp.minimum(row0, N - CHP)
                jlo = row0 - base
                jhi = jnp.minimum(end, base + CHP) - base
                pltpu.sync_copy(h_hbm.at[pl.ds(base, CHP)], hbuf_v)

                def row_body(j, rc):
                    sums = []
                    maxs = []
                    for k in range(8):
                        v = hbuf_v[j, pl.ds(k * 16, 16)]
                        sums.append(rc[k] + v)
                        maxs.append(jnp.maximum(rc[8 + k], v))
                    return tuple(sums) + tuple(maxs)

                return lax.fori_loop(jlo, jhi, row_body, carry)

            init = tuple(jnp.zeros((16,), jnp.float32) for _ in range(8)) \
                + tuple(jnp.full((16,), -jnp.inf, jnp.float32)
                        for _ in range(8))
            accs = lax.fori_loop(0, nch, chunk_body, init)
            for k in range(8):
                sstage_v[pl.ds(ti * D + k * 16, 16)] = accs[k]
                mstage_v[pl.ds(ti * D + k * 16, 16)] = accs[8 + k]
        pltpu.sync_copy(sstage_v, sums_hbm.at[g])
        pltpu.sync_copy(mstage_v, maxs_hbm.at[g])


def _dinv_from_degp(degp):
    deg = degp[0, :, :1] + degp[1, :, :1] + 1.0
    return lax.rsqrt(deg)


def _tc_pre_body(x_ref, w_ref, degp_ref, hws_ref):
    dinv = _dinv_from_degp(degp_ref[...])
    hw = jnp.dot(x_ref[...], w_ref[...], preferred_element_type=jnp.float32)
    hws_ref[...] = hw * dinv


def _tc_layer_body(sp_ref, hws_ref, degp_ref, b_ref, g_ref, bb_ref, wn_ref,
                   h_ref, hwsn_ref):
    dinv = _dinv_from_degp(degp_ref[...])
    s = sp_ref[0] + sp_ref[1]
    pre = dinv * (s + hws_ref[...]) + b_ref[...]
    mu = jnp.mean(pre, axis=0, keepdims=True)
    var = jnp.mean((pre - mu) ** 2, axis=0, keepdims=True)
    h = (pre - mu) * lax.rsqrt(var + EPS) * g_ref[...] + bb_ref[...]
    h_ref[...] = h
    if hwsn_ref is not None:
        hwsn_ref[...] = jnp.dot(
            h, wn_ref[...], preferred_element_type=jnp.float32) * dinv


def _tc_layer_last_body(sp_ref, hws_ref, degp_ref, b_ref, g_ref, bb_ref,
                        batch_ref, h_ref, offs_ref):
    _tc_layer_body(sp_ref, hws_ref, degp_ref, b_ref, g_ref, bb_ref, None,
                   h_ref, None)
    bcol = batch_ref[...]
    seg = lax.broadcasted_iota(jnp.int32, (N, B), 1)
    m = (bcol == seg).astype(jnp.float32)
    cnt = jnp.sum(m, axis=0)
    tri = (lax.broadcasted_iota(jnp.int32, (B, B), 0)
           < lax.broadcasted_iota(jnp.int32, (B, B), 1)).astype(jnp.float32)
    offs = lax.dot_general(cnt, tri, (((0,), (0,)), ((), ())),
                           preferred_element_type=jnp.float32)
    offs_full = jnp.concatenate(
        [offs, jnp.full((NOFF - B,), float(N), jnp.float32)])
    offs_ref[...] = offs_full.astype(jnp.int32)


def _tc_head_body(sums_ref, maxs_ref, offs_ref, bg_ref, bbo_ref,
                  w1_ref, b1_ref, w2_ref, b2_ref, w3_ref, b3_ref,
                  out_ref):
    sums = sums_ref[...]
    maxs = maxs_ref[...]
    offs = offs_ref[...].astype(jnp.float32)
    cnt = offs[1:B + 1] - offs[0:B]
    avg = sums / jnp.maximum(cnt, 1.0)[:, None]
    hp = jnp.concatenate([avg, sums, maxs], axis=1)
    mu = jnp.mean(hp, axis=0, keepdims=True)
    var = jnp.mean((hp - mu) ** 2, axis=0, keepdims=True)
    z = (hp - mu) * lax.rsqrt(var + EPS) * bg_ref[...] + bbo_ref[...]
    z = jnp.maximum(
        jnp.dot(z, w1_ref[...], preferred_element_type=jnp.float32)
        + b1_ref[...], 0.0)
    z = jnp.maximum(
        jnp.dot(z, w2_ref[...], preferred_element_type=jnp.float32)
        + b2_ref[...], 0.0)
    zl = jnp.dot(z, w3_ref[...], preferred_element_type=jnp.float32) \
        + b3_ref[...]
    zmax = jnp.max(zl, axis=1, keepdims=True)
    lse = zmax + jnp.log(jnp.sum(jnp.exp(zl - zmax), axis=1, keepdims=True))
    out_ref[...] = zl - lse


_f32 = jnp.float32


def _call_tc(body, out_shape, *args, scratch_shapes=()):
    return pl.pallas_call(
        body,
        out_shape=out_shape,
        scratch_shapes=list(scratch_shapes),
    )(*args)


def kernel(x, edge_index, batch, conv_W, conv_b, bn_g, bn_b, bno_g, bno_b,
           W1, b1, W2, b2, W3, b3):
    zeros16 = jnp.zeros((RQ, 16), _f32)
    ones16 = jnp.ones((CH, 16), _f32)
    zerosd = jnp.zeros((RQ, D), _f32)

    degp = _sc_deg(edge_index, zeros16, ones16)

    hws = _call_tc(_tc_pre_body, jax.ShapeDtypeStruct((N, D), _f32),
                   x, conv_W[0], degp)

    hs = []
    for i in range(2):
        sp = _sc_agg(hws, edge_index, zerosd)
        h, hws = _call_tc(
            _tc_layer_body,
            (jax.ShapeDtypeStruct((N, D), _f32),
             jax.ShapeDtypeStruct((N, D), _f32)),
            sp, hws, degp, conv_b[i].reshape(1, D), bn_g[i].reshape(1, D),
            bn_b[i].reshape(1, D), conv_W[i + 1])
        hs.append(h)

    sp = _sc_agg(hws, edge_index, zerosd)
    h3, offs = _call_tc(
        _tc_layer_last_body,
        (jax.ShapeDtypeStruct((N, D), _f32),
         jax.ShapeDtypeStruct((NOFF,), jnp.int32)),
        sp, hws, degp, conv_b[2].reshape(1, D), bn_g[2].reshape(1, D),
        bn_b[2].reshape(1, D), batch.reshape(N, 1))
    sums, maxs = _sc_pool(hs[0], hs[1], h3, offs)
    out = _call_tc(
        _tc_head_body, jax.ShapeDtypeStruct((B, 10), _f32),
        sums, maxs, offs,
        bno_g.reshape(1, D_POOL), bno_b.reshape(1, D_POOL),
        W1, b1.reshape(1, -1), W2, b2.reshape(1, -1), W3, b3.reshape(1, -1))
    return out
